# Initial kernel scaffold; baseline (speedup 1.0000x reference)
import jax
import jax.numpy as jnp
from jax.experimental import pallas as pl


def kernel(x, edge_index, W_self1, W_neigh1, W_self2, W_neigh2):
    raise NotImplementedError("write your pallas kernel here")


# speedup vs baseline: 5.4225x; 5.4225x over previous
import functools

import jax
import jax.numpy as jnp
from jax import lax
from jax.experimental import pallas as pl
from jax.experimental.pallas import tpu as pltpu
from jax.experimental.pallas import tpu_sc as plsc

N_NODES = 10000
D = 128
N_EDGES = 320000
NC = 2
NS = 16
NW = NC * NS
EPW = N_EDGES // NW
CHUNK = 80
NCHUNK = EPW // CHUNK
NPAD = 10240
RPT = NPAD // NS
ZROWS = 64


def _sc_agg_body(h_hbm, src_hbm, dst_hbm, acc_out, deg_out,
                 src_c, dst_c, rows, deg_local, zbuf, acc_sh, sem):
    c = lax.axis_index("c")
    s = lax.axis_index("s")
    w = c * NS + s

    zeros16 = jnp.zeros((16,), jnp.float32)
    ones16 = jnp.ones((16,), jnp.float32)

    def _zb(r, carry):
        for j in range(D // 16):
            zbuf[r, pl.ds(j * 16, 16)] = zeros16
        return carry
    lax.fori_loop(0, ZROWS, _zb, 0)

    def _zd(r, carry):
        deg_local[pl.ds(r * 16, 16)] = zeros16
        return carry
    lax.fori_loop(0, N_NODES // 16, _zd, 0)

    for j in range(RPT // ZROWS):
        pltpu.sync_copy(zbuf, acc_sh.at[pl.ds(s * RPT + j * ZROWS, ZROWS)])
    plsc.subcore_barrier()

    def _chunk(i, carry):
        base = w * EPW + i * CHUNK
        pltpu.sync_copy(src_hbm.at[pl.ds(base, CHUNK)], src_c)
        pltpu.sync_copy(dst_hbm.at[pl.ds(base, CHUNK)], dst_c)
        pltpu.async_copy(h_hbm.at[src_c], rows, sem).wait()
        pltpu.sync_copy(rows, acc_sh.at[dst_c], add=True)
        for j in range(CHUNK // 16):
            idx = dst_c[pl.ds(j * 16, 16)]
            plsc.addupdate_scatter(deg_local, [idx], ones16)
        return carry
    lax.fori_loop(0, NCHUNK, _chunk, 0)

    plsc.subcore_barrier()

    for j in range(RPT // ZROWS):
        r0 = s * RPT + j * ZROWS
        pltpu.sync_copy(acc_sh.at[pl.ds(r0, ZROWS)], zbuf)
        pltpu.sync_copy(zbuf, acc_out.at[pl.ds(c * NPAD + r0, ZROWS)])
    pltpu.sync_copy(deg_local, deg_out.at[pl.ds(w * N_NODES, N_NODES)])


_sc_agg = pl.kernel(
    _sc_agg_body,
    mesh=plsc.VectorSubcoreMesh(core_axis_name="c", subcore_axis_name="s"),
    out_type=[
        jax.ShapeDtypeStruct((NC * NPAD, D), jnp.float32),
        jax.ShapeDtypeStruct((NW * N_NODES,), jnp.float32),
    ],
    scratch_types=[
        pltpu.VMEM((CHUNK,), jnp.int32),
        pltpu.VMEM((CHUNK,), jnp.int32),
        pltpu.VMEM((CHUNK, D), jnp.float32),
        pltpu.VMEM((N_NODES,), jnp.float32),
        pltpu.VMEM((ZROWS, D), jnp.float32),
        pltpu.VMEM_SHARED((NPAD, D), jnp.float32),
        pltpu.SemaphoreType.DMA,
    ],
    compiler_params=pltpu.CompilerParams(needs_layout_passes=False),
)


def _tc_layer_body(relu, x_ref, acc_ref, deg_ref, ws_ref, wn_ref, o_ref):
    acc = acc_ref[0] + acc_ref[1]
    deg = jnp.sum(deg_ref[...], axis=1)
    inv = 1.0 / jnp.clip(deg, 1.0, None)
    hn = acc * inv[:, None]
    y = (jnp.dot(x_ref[...], ws_ref[...],
                 preferred_element_type=jnp.float32,
                 precision=lax.Precision.HIGHEST)
         + jnp.dot(hn, wn_ref[...],
                   preferred_element_type=jnp.float32,
                   precision=lax.Precision.HIGHEST))
    o_ref[...] = jnp.maximum(y, 0.0) if relu else y


def _tc_layer(x, accp, deg32t, W_self, W_neigh, relu):
    blk = 1000
    grid = N_NODES // blk
    return pl.pallas_call(
        functools.partial(_tc_layer_body, relu),
        grid=(grid,),
        in_specs=[
            pl.BlockSpec((blk, D), lambda i: (i, 0)),
            pl.BlockSpec((NC, blk, D), lambda i: (0, i, 0)),
            pl.BlockSpec((blk, NW), lambda i: (i, 0)),
            pl.BlockSpec((D, D), lambda i: (0, 0)),
            pl.BlockSpec((D, D), lambda i: (0, 0)),
        ],
        out_specs=pl.BlockSpec((blk, D), lambda i: (i, 0)),
        out_shape=jax.ShapeDtypeStruct((N_NODES, D), jnp.float32),
    )(x, accp, deg32t, W_self, W_neigh)


def kernel(x, edge_index, W_self1, W_neigh1, W_self2, W_neigh2):
    src = edge_index[0].astype(jnp.int32)
    dst = edge_index[1].astype(jnp.int32)
    accp1, degf = _sc_agg(x, src, dst)
    deg32t = degf.reshape(NW, N_NODES).T
    h1 = _tc_layer(x, accp1.reshape(NC, NPAD, D), deg32t, W_self1, W_neigh1, True)
    accp2, _ = _sc_agg(h1, src, dst)
    return _tc_layer(h1, accp2.reshape(NC, NPAD, D), deg32t, W_self2, W_neigh2, False)


# speedup vs baseline: 9.9149x; 1.8285x over previous
import functools

import jax
import jax.numpy as jnp
from jax import lax
from jax.experimental import pallas as pl
from jax.experimental.pallas import tpu as pltpu
from jax.experimental.pallas import tpu_sc as plsc

N_NODES = 10000
D = 128
N_EDGES = 320000
NC = 2
NS = 16
NW = NC * NS
EPW = N_EDGES // NW
CHUNK = 80
NCHUNK = EPW // CHUNK
NPAD = 10240
RPT = NPAD // NS
ZROWS = 32
NBUF = 3


def _sc_agg_body(h_hbm, src_hbm, dst_hbm, acc_out, deg_out,
                 src_b, dst_b, rows_b, deg_local, zbuf, acc_sh, gsem, ssem):
    c = lax.axis_index("c")
    s = lax.axis_index("s")
    w = c * NS + s

    zeros16 = jnp.zeros((16,), jnp.float32)
    ones16 = jnp.ones((16,), jnp.float32)

    def _zb(r, carry):
        for j in range(D // 16):
            zbuf[r, pl.ds(j * 16, 16)] = zeros16
        return carry
    lax.fori_loop(0, ZROWS, _zb, 0)

    def _zd(r, carry):
        deg_local[pl.ds(r * 16, 16)] = zeros16
        return carry
    lax.fori_loop(0, N_NODES // 16, _zd, 0)

    for j in range(RPT // ZROWS):
        pltpu.sync_copy(zbuf, acc_sh.at[pl.ds(s * RPT + j * ZROWS, ZROWS)])
    plsc.subcore_barrier()

    def _stage_and_gather(i, b):
        base = w * EPW + i * CHUNK
        pltpu.sync_copy(src_hbm.at[pl.ds(base, CHUNK)], src_b.at[b])
        pltpu.sync_copy(dst_hbm.at[pl.ds(base, CHUNK)], dst_b.at[b])
        pltpu.async_copy(h_hbm.at[src_b.at[b]], rows_b.at[b], gsem)

    def _step(i, p):
        q = (p + 1) % NBUF

        @pl.when(i >= NBUF - 1)
        def _():
            pltpu.make_async_copy(rows_b.at[q], acc_sh.at[dst_b.at[q]],
                                  ssem).wait()

        @pl.when(i + 1 < NCHUNK)
        def _():
            _stage_and_gather(i + 1, q)

        pltpu.make_async_copy(h_hbm.at[src_b.at[p]], rows_b.at[p],
                              gsem).wait()
        pltpu.async_copy(rows_b.at[p], acc_sh.at[dst_b.at[p]], ssem,
                         add=True)
        for j in range(CHUNK // 16):
            idx = dst_b[p, pl.ds(j * 16, 16)]
            plsc.addupdate_scatter(deg_local, [idx], ones16)

    _stage_and_gather(0, 0)

    def _chunk(i, carry):
        for p in range(NBUF):
            @pl.when(lax.rem(i, NBUF) == p)
            def _(i=i, p=p):
                _step(i, p)
        return carry
    lax.fori_loop(0, NCHUNK, _chunk, 0)

    for i in (NCHUNK - 2, NCHUNK - 1):
        b = i % NBUF
        pltpu.make_async_copy(rows_b.at[b], acc_sh.at[dst_b.at[b]],
                              ssem).wait()

    plsc.subcore_barrier()

    for j in range(RPT // ZROWS):
        r0 = s * RPT + j * ZROWS
        pltpu.sync_copy(acc_sh.at[pl.ds(r0, ZROWS)], zbuf)
        pltpu.sync_copy(zbuf, acc_out.at[pl.ds(c * NPAD + r0, ZROWS)])
    pltpu.sync_copy(deg_local, deg_out.at[pl.ds(w * N_NODES, N_NODES)])


_sc_agg = pl.kernel(
    _sc_agg_body,
    mesh=plsc.VectorSubcoreMesh(core_axis_name="c", subcore_axis_name="s"),
    out_type=[
        jax.ShapeDtypeStruct((NC * NPAD, D), jnp.float32),
        jax.ShapeDtypeStruct((NW * N_NODES,), jnp.float32),
    ],
    scratch_types=[
        pltpu.VMEM((NBUF, CHUNK), jnp.int32),
        pltpu.VMEM((NBUF, CHUNK), jnp.int32),
        pltpu.VMEM((NBUF, CHUNK, D), jnp.float32),
        pltpu.VMEM((N_NODES,), jnp.float32),
        pltpu.VMEM((ZROWS, D), jnp.float32),
        pltpu.VMEM_SHARED((NPAD, D), jnp.float32),
        pltpu.SemaphoreType.DMA,
        pltpu.SemaphoreType.DMA,
    ],
    compiler_params=pltpu.CompilerParams(needs_layout_passes=False),
)


def _tc_layer_body(relu, x_ref, acc_ref, deg_ref, ws_ref, wn_ref, o_ref):
    acc = acc_ref[0] + acc_ref[1]
    deg = jnp.sum(deg_ref[...], axis=1)
    inv = 1.0 / jnp.clip(deg, 1.0, None)
    hn = acc * inv[:, None]
    y = (jnp.dot(x_ref[...], ws_ref[...],
                 preferred_element_type=jnp.float32,
                 precision=lax.Precision.HIGHEST)
         + jnp.dot(hn, wn_ref[...],
                   preferred_element_type=jnp.float32,
                   precision=lax.Precision.HIGHEST))
    o_ref[...] = jnp.maximum(y, 0.0) if relu else y


def _tc_layer(x, accp, deg32t, W_self, W_neigh, relu):
    blk = 1000
    grid = N_NODES // blk
    return pl.pallas_call(
        functools.partial(_tc_layer_body, relu),
        grid=(grid,),
        in_specs=[
            pl.BlockSpec((blk, D), lambda i: (i, 0)),
            pl.BlockSpec((NC, blk, D), lambda i: (0, i, 0)),
            pl.BlockSpec((blk, NW), lambda i: (i, 0)),
            pl.BlockSpec((D, D), lambda i: (0, 0)),
            pl.BlockSpec((D, D), lambda i: (0, 0)),
        ],
        out_specs=pl.BlockSpec((blk, D), lambda i: (i, 0)),
        out_shape=jax.ShapeDtypeStruct((N_NODES, D), jnp.float32),
    )(x, accp, deg32t, W_self, W_neigh)


def kernel(x, edge_index, W_self1, W_neigh1, W_self2, W_neigh2):
    src = edge_index[0].astype(jnp.int32)
    dst = edge_index[1].astype(jnp.int32)
    accp1, degf = _sc_agg(x, src, dst)
    deg32t = degf.reshape(NW, N_NODES).T
    h1 = _tc_layer(x, accp1.reshape(NC, NPAD, D), deg32t, W_self1, W_neigh1, True)
    accp2, _ = _sc_agg(h1, src, dst)
    return _tc_layer(h1, accp2.reshape(NC, NPAD, D), deg32t, W_self2, W_neigh2, False)


# speedup vs baseline: 11.0871x; 1.1182x over previous
import functools

import jax
import jax.numpy as jnp
from jax import lax
from jax.experimental import pallas as pl
from jax.experimental.pallas import tpu as pltpu
from jax.experimental.pallas import tpu_sc as plsc

N_NODES = 10000
D = 128
N_EDGES = 320000
NC = 2
NS = 16
NW = NC * NS
EPW = N_EDGES // NW
CHUNK = 80
NCHUNK = EPW // CHUNK
NPAD = 10240
RPT = NPAD // NS
ZROWS = 32
NBUF = 3


def _sc_agg_body(h_hbm, sd_hbm, acc_out, deg_out,
                 sd_b, rows_b, deg_local, zbuf, acc_sh, gsem, ssem):
    c = lax.axis_index("c")
    s = lax.axis_index("s")
    w = c * NS + s

    zeros16 = jnp.zeros((16,), jnp.float32)
    ones16 = jnp.ones((16,), jnp.float32)

    def _zb(r, carry):
        for j in range(D // 16):
            zbuf[r, pl.ds(j * 16, 16)] = zeros16
        return carry
    lax.fori_loop(0, ZROWS, _zb, 0)

    def _zd(r, carry):
        deg_local[pl.ds(r * 16, 16)] = zeros16
        return carry
    lax.fori_loop(0, N_NODES // 16, _zd, 0)

    for j in range(RPT // ZROWS):
        pltpu.sync_copy(zbuf, acc_sh.at[pl.ds(s * RPT + j * ZROWS, ZROWS)])
    plsc.subcore_barrier()

    def _stage_and_gather(i, b):
        pltpu.sync_copy(sd_hbm.at[w * NCHUNK + i], sd_b.at[b])
        pltpu.async_copy(h_hbm.at[sd_b.at[b, 0]], rows_b.at[b], gsem)

    def _step(i, p):
        q = (p + 1) % NBUF

        @pl.when(i >= NBUF - 1)
        def _():
            pltpu.make_async_copy(rows_b.at[q], acc_sh.at[sd_b.at[q, 1]],
                                  ssem).wait()

        @pl.when(i + 1 < NCHUNK)
        def _():
            _stage_and_gather(i + 1, q)

        pltpu.make_async_copy(h_hbm.at[sd_b.at[p, 0]], rows_b.at[p],
                              gsem).wait()
        pltpu.async_copy(rows_b.at[p], acc_sh.at[sd_b.at[p, 1]], ssem,
                         add=True)
        for j in range(CHUNK // 16):
            idx = sd_b[p, 1, pl.ds(j * 16, 16)]
            plsc.addupdate_scatter(deg_local, [idx], ones16)

    _stage_and_gather(0, 0)

    def _chunk(i, carry):
        for p in range(NBUF):
            @pl.when(lax.rem(i, NBUF) == p)
            def _(i=i, p=p):
                _step(i, p)
        return carry
    lax.fori_loop(0, NCHUNK, _chunk, 0)

    for i in (NCHUNK - 2, NCHUNK - 1):
        b = i % NBUF
        pltpu.make_async_copy(rows_b.at[b], acc_sh.at[sd_b.at[b, 1]],
                              ssem).wait()

    plsc.subcore_barrier()

    for j in range(RPT // ZROWS):
        r0 = s * RPT + j * ZROWS
        pltpu.sync_copy(acc_sh.at[pl.ds(r0, ZROWS)], zbuf)
        pltpu.sync_copy(zbuf, acc_out.at[pl.ds(c * NPAD + r0, ZROWS)])
    pltpu.sync_copy(deg_local, deg_out.at[pl.ds(w * N_NODES, N_NODES)])


_sc_agg = pl.kernel(
    _sc_agg_body,
    mesh=plsc.VectorSubcoreMesh(core_axis_name="c", subcore_axis_name="s"),
    out_type=[
        jax.ShapeDtypeStruct((NC * NPAD, D), jnp.float32),
        jax.ShapeDtypeStruct((NW * N_NODES,), jnp.float32),
    ],
    scratch_types=[
        pltpu.VMEM((NBUF, 2, CHUNK), jnp.int32),
        pltpu.VMEM((NBUF, CHUNK, D), jnp.float32),
        pltpu.VMEM((N_NODES,), jnp.float32),
        pltpu.VMEM((ZROWS, D), jnp.float32),
        pltpu.VMEM_SHARED((NPAD, D), jnp.float32),
        pltpu.SemaphoreType.DMA,
        pltpu.SemaphoreType.DMA,
    ],
    compiler_params=pltpu.CompilerParams(needs_layout_passes=False),
)


def _tc_layer_body(relu, x_ref, acc_ref, deg_ref, ws_ref, wn_ref, o_ref):
    acc = acc_ref[0] + acc_ref[1]
    deg = jnp.sum(deg_ref[...], axis=1)
    inv = 1.0 / jnp.clip(deg, 1.0, None)
    hn = acc * inv[:, None]
    y = (jnp.dot(x_ref[...], ws_ref[...],
                 preferred_element_type=jnp.float32,
                 precision=lax.Precision.HIGHEST)
         + jnp.dot(hn, wn_ref[...],
                   preferred_element_type=jnp.float32,
                   precision=lax.Precision.HIGHEST))
    o_ref[...] = jnp.maximum(y, 0.0) if relu else y


def _tc_layer(x, accp, deg32t, W_self, W_neigh, relu):
    blk = 1000
    grid = N_NODES // blk
    return pl.pallas_call(
        functools.partial(_tc_layer_body, relu),
        grid=(grid,),
        in_specs=[
            pl.BlockSpec((blk, D), lambda i: (i, 0)),
            pl.BlockSpec((NC, blk, D), lambda i: (0, i, 0)),
            pl.BlockSpec((blk, NW), lambda i: (i, 0)),
            pl.BlockSpec((D, D), lambda i: (0, 0)),
            pl.BlockSpec((D, D), lambda i: (0, 0)),
        ],
        out_specs=pl.BlockSpec((blk, D), lambda i: (i, 0)),
        out_shape=jax.ShapeDtypeStruct((N_NODES, D), jnp.float32),
    )(x, accp, deg32t, W_self, W_neigh)


def kernel(x, edge_index, W_self1, W_neigh1, W_self2, W_neigh2):
    ei = edge_index.astype(jnp.int32).reshape(2, NW * NCHUNK, CHUNK)
    sd = jnp.swapaxes(ei, 0, 1)
    accp1, degf = _sc_agg(x, sd)
    deg32t = degf.reshape(NW, N_NODES).T
    h1 = _tc_layer(x, accp1.reshape(NC, NPAD, D), deg32t, W_self1, W_neigh1, True)
    accp2, _ = _sc_agg(h1, sd)
    return _tc_layer(h1, accp2.reshape(NC, NPAD, D), deg32t, W_self2, W_neigh2, False)


# speedup vs baseline: 13.1068x; 1.1822x over previous
import functools

import jax
import jax.numpy as jnp
from jax import lax
from jax.experimental import pallas as pl
from jax.experimental.pallas import tpu as pltpu
from jax.experimental.pallas import tpu_sc as plsc

N_NODES = 10000
D = 128
N_EDGES = 320000
NC = 2
NS = 16
NW = NC * NS
EPW = N_EDGES // NW
CHUNK = 80
NCHUNK = EPW // CHUNK
NPAD = 10240
RPT = NPAD // NS
ZROWS = 32
NBUF = 3
IBUF = 4


def _sc_agg_body(h_hbm, sd_hbm, acc_out, deg_out,
                 sd_b, rows_b, deg_local, zbuf, acc_sh, gsem, ssem, isem):
    c = lax.axis_index("c")
    s = lax.axis_index("s")
    w = c * NS + s

    zeros16 = jnp.zeros((16,), jnp.float32)
    ones16 = jnp.ones((16,), jnp.float32)

    def _zb(r, carry):
        for j in range(D // 16):
            zbuf[r, pl.ds(j * 16, 16)] = zeros16
        return carry
    lax.fori_loop(0, ZROWS, _zb, 0)

    def _zd(r, carry):
        deg_local[pl.ds(r * 16, 16)] = zeros16
        return carry
    lax.fori_loop(0, N_NODES // 16, _zd, 0)

    for j in range(RPT // ZROWS):
        pltpu.sync_copy(zbuf, acc_sh.at[pl.ds(s * RPT + j * ZROWS, ZROWS)])
    plsc.subcore_barrier()

    def _stage(i, b4):
        pltpu.async_copy(sd_hbm.at[w * NCHUNK + i], sd_b.at[b4], isem)

    def _step(i, p3, p4):
        q3 = (p3 + 1) % NBUF
        q4 = (p4 + 1) % IBUF
        r4 = (p4 + 2) % IBUF

        @pl.when(i >= 2)
        def _():
            pltpu.make_async_copy(rows_b.at[q3], acc_sh.at[sd_b.at[r4, 1]],
                                  ssem).wait()

        @pl.when(i + 2 < NCHUNK)
        def _():
            _stage(i + 2, r4)

        @pl.when(i + 1 < NCHUNK)
        def _():
            pltpu.make_async_copy(sd_hbm.at[w * NCHUNK + i + 1],
                                  sd_b.at[q4], isem).wait()
            pltpu.async_copy(h_hbm.at[sd_b.at[q4, 0]], rows_b.at[q3], gsem)

        for j in range(CHUNK // 16):
            idx = sd_b[p4, 1, pl.ds(j * 16, 16)]
            plsc.addupdate_scatter(deg_local, [idx], ones16)

        pltpu.make_async_copy(h_hbm.at[sd_b.at[p4, 0]], rows_b.at[p3],
                              gsem).wait()
        pltpu.async_copy(rows_b.at[p3], acc_sh.at[sd_b.at[p4, 1]], ssem,
                         add=True)

    _stage(0, 0)
    _stage(1, 1)
    pltpu.make_async_copy(sd_hbm.at[w * NCHUNK], sd_b.at[0], isem).wait()
    pltpu.async_copy(h_hbm.at[sd_b.at[0, 0]], rows_b.at[0], gsem)

    NPAR = NBUF * IBUF
    def _chunk(i, carry):
        for p in range(NPAR):
            @pl.when(lax.rem(i, NPAR) == p)
            def _(i=i, p=p):
                _step(i, p % NBUF, p % IBUF)
        return carry
    lax.fori_loop(0, NCHUNK, _chunk, 0)

    for i in (NCHUNK - 2, NCHUNK - 1):
        pltpu.make_async_copy(rows_b.at[i % NBUF],
                              acc_sh.at[sd_b.at[i % IBUF, 1]], ssem).wait()

    plsc.subcore_barrier()

    for j in range(RPT // ZROWS):
        r0 = s * RPT + j * ZROWS
        pltpu.sync_copy(acc_sh.at[pl.ds(r0, ZROWS)], zbuf)
        pltpu.sync_copy(zbuf, acc_out.at[pl.ds(c * NPAD + r0, ZROWS)])
    pltpu.sync_copy(deg_local, deg_out.at[pl.ds(w * N_NODES, N_NODES)])


_sc_agg = pl.kernel(
    _sc_agg_body,
    mesh=plsc.VectorSubcoreMesh(core_axis_name="c", subcore_axis_name="s"),
    out_type=[
        jax.ShapeDtypeStruct((NC * NPAD, D), jnp.float32),
        jax.ShapeDtypeStruct((NW * N_NODES,), jnp.float32),
    ],
    scratch_types=[
        pltpu.VMEM((IBUF, 2, CHUNK), jnp.int32),
        pltpu.VMEM((NBUF, CHUNK, D), jnp.float32),
        pltpu.VMEM((N_NODES,), jnp.float32),
        pltpu.VMEM((ZROWS, D), jnp.float32),
        pltpu.VMEM_SHARED((NPAD, D), jnp.float32),
        pltpu.SemaphoreType.DMA,
        pltpu.SemaphoreType.DMA,
        pltpu.SemaphoreType.DMA,
    ],
    compiler_params=pltpu.CompilerParams(needs_layout_passes=False),
)


def _tc_layer_body(relu, x_ref, acc_ref, deg_ref, ws_ref, wn_ref, o_ref):
    acc = acc_ref[0] + acc_ref[1]
    deg = jnp.sum(deg_ref[...], axis=1)
    inv = 1.0 / jnp.clip(deg, 1.0, None)
    hn = acc * inv[:, None]
    y = (jnp.dot(x_ref[...], ws_ref[...],
                 preferred_element_type=jnp.float32,
                 precision=lax.Precision.HIGHEST)
         + jnp.dot(hn, wn_ref[...],
                   preferred_element_type=jnp.float32,
                   precision=lax.Precision.HIGHEST))
    o_ref[...] = jnp.maximum(y, 0.0) if relu else y


def _tc_layer(x, accp, deg32t, W_self, W_neigh, relu):
    blk = 1000
    grid = N_NODES // blk
    return pl.pallas_call(
        functools.partial(_tc_layer_body, relu),
        grid=(grid,),
        in_specs=[
            pl.BlockSpec((blk, D), lambda i: (i, 0)),
            pl.BlockSpec((NC, blk, D), lambda i: (0, i, 0)),
            pl.BlockSpec((blk, NW), lambda i: (i, 0)),
            pl.BlockSpec((D, D), lambda i: (0, 0)),
            pl.BlockSpec((D, D), lambda i: (0, 0)),
        ],
        out_specs=pl.BlockSpec((blk, D), lambda i: (i, 0)),
        out_shape=jax.ShapeDtypeStruct((N_NODES, D), jnp.float32),
    )(x, accp, deg32t, W_self, W_neigh)


def kernel(x, edge_index, W_self1, W_neigh1, W_self2, W_neigh2):
    ei = edge_index.astype(jnp.int32).reshape(2, NW * NCHUNK, CHUNK)
    sd = jnp.swapaxes(ei, 0, 1)
    accp1, degf = _sc_agg(x, sd)
    deg32t = degf.reshape(NW, N_NODES).T
    h1 = _tc_layer(x, accp1.reshape(NC, NPAD, D), deg32t, W_self1, W_neigh1, True)
    accp2, _ = _sc_agg(h1, sd)
    return _tc_layer(h1, accp2.reshape(NC, NPAD, D), deg32t, W_self2, W_neigh2, False)


# speedup vs baseline: 14.0576x; 1.0725x over previous
import functools

import jax
import jax.numpy as jnp
from jax import lax
from jax.experimental import pallas as pl
from jax.experimental.pallas import tpu as pltpu
from jax.experimental.pallas import tpu_sc as plsc

N_NODES = 10000
D = 128
N_EDGES = 320000
NC = 2
NS = 16
NW = NC * NS
EPW = N_EDGES // NW
CHUNK = 80
NCHUNK = EPW // CHUNK
NPAD = 10240
RPT = NPAD // NS
ZROWS = 32
NBUF = 3
IBUF = 4


def _sc_agg_body(h_hbm, sd_hbm, acc_out, deg_out,
                 sd_b, rows_b, deg_local, zbuf, acc_sh, gsem, ssem, isem):
    c = lax.axis_index("c")
    s = lax.axis_index("s")
    w = c * NS + s

    zeros16 = jnp.zeros((16,), jnp.float32)
    ones16 = jnp.ones((16,), jnp.float32)

    def _zb(r, carry):
        for j in range(D // 16):
            zbuf[r, pl.ds(j * 16, 16)] = zeros16
        return carry
    lax.fori_loop(0, ZROWS, _zb, 0)

    def _zd(r, carry):
        deg_local[pl.ds(r * 16, 16)] = zeros16
        return carry
    lax.fori_loop(0, N_NODES // 16, _zd, 0)

    for j in range(RPT // ZROWS):
        pltpu.sync_copy(zbuf, acc_sh.at[pl.ds(s * RPT + j * ZROWS, ZROWS)])
    plsc.subcore_barrier()

    def _stage(i, b4):
        pltpu.async_copy(sd_hbm.at[w * NCHUNK + i], sd_b.at[b4], isem)

    def _step(i, p3, p4):
        q3 = (p3 + 1) % NBUF
        q4 = (p4 + 1) % IBUF
        r4 = (p4 + 2) % IBUF

        @pl.when(i >= 2)
        def _():
            pltpu.make_async_copy(rows_b.at[q3], acc_sh.at[sd_b.at[r4, 1]],
                                  ssem).wait()

        @pl.when(i + 2 < NCHUNK)
        def _():
            _stage(i + 2, r4)

        @pl.when(i + 1 < NCHUNK)
        def _():
            pltpu.make_async_copy(sd_hbm.at[w * NCHUNK + i + 1],
                                  sd_b.at[q4], isem).wait()
            pltpu.async_copy(h_hbm.at[sd_b.at[q4, 0]], rows_b.at[q3], gsem)

        for j in range(CHUNK // 16):
            idx = sd_b[p4, 1, pl.ds(j * 16, 16)]
            plsc.addupdate_scatter(deg_local, [idx], ones16)

        pltpu.make_async_copy(h_hbm.at[sd_b.at[p4, 0]], rows_b.at[p3],
                              gsem).wait()
        pltpu.async_copy(rows_b.at[p3], acc_sh.at[sd_b.at[p4, 1]], ssem,
                         add=True)

    _stage(0, 0)
    _stage(1, 1)
    pltpu.make_async_copy(sd_hbm.at[w * NCHUNK], sd_b.at[0], isem).wait()
    pltpu.async_copy(h_hbm.at[sd_b.at[0, 0]], rows_b.at[0], gsem)

    NPAR = NBUF * IBUF
    def _chunk(i, carry):
        for p in range(NPAR):
            @pl.when(lax.rem(i, NPAR) == p)
            def _(i=i, p=p):
                _step(i, p % NBUF, p % IBUF)
        return carry
    lax.fori_loop(0, NCHUNK, _chunk, 0)

    for i in (NCHUNK - 2, NCHUNK - 1):
        pltpu.make_async_copy(rows_b.at[i % NBUF],
                              acc_sh.at[sd_b.at[i % IBUF, 1]], ssem).wait()

    plsc.subcore_barrier()

    for j in range(RPT // ZROWS):
        r0 = s * RPT + j * ZROWS
        pltpu.sync_copy(acc_sh.at[pl.ds(r0, ZROWS)], zbuf)
        pltpu.sync_copy(zbuf, acc_out.at[pl.ds(c * NPAD + r0, ZROWS)])
    pltpu.sync_copy(deg_local, deg_out.at[pl.ds(w * N_NODES, N_NODES)])


_sc_agg = pl.kernel(
    _sc_agg_body,
    mesh=plsc.VectorSubcoreMesh(core_axis_name="c", subcore_axis_name="s"),
    out_type=[
        jax.ShapeDtypeStruct((NC * NPAD, D), jnp.float32),
        jax.ShapeDtypeStruct((NW * N_NODES,), jnp.float32),
    ],
    scratch_types=[
        pltpu.VMEM((IBUF, 2, CHUNK), jnp.int32),
        pltpu.VMEM((NBUF, CHUNK, D), jnp.float32),
        pltpu.VMEM((N_NODES,), jnp.float32),
        pltpu.VMEM((ZROWS, D), jnp.float32),
        pltpu.VMEM_SHARED((NPAD, D), jnp.float32),
        pltpu.SemaphoreType.DMA,
        pltpu.SemaphoreType.DMA,
        pltpu.SemaphoreType.DMA,
    ],
    compiler_params=pltpu.CompilerParams(needs_layout_passes=False),
)


BLK = 1280
TGRID = (N_NODES + BLK - 1) // BLK


def _tc_self_body(x_ref, w_ref, o_ref):
    o_ref[...] = jnp.dot(x_ref[...], w_ref[...],
                         preferred_element_type=jnp.float32,
                         precision=lax.Precision.HIGHEST)


def _tc_self(x, W):
    return pl.pallas_call(
        _tc_self_body,
        grid=(TGRID,),
        in_specs=[
            pl.BlockSpec((BLK, D), lambda i: (i, 0)),
            pl.BlockSpec((D, D), lambda i: (0, 0)),
        ],
        out_specs=pl.BlockSpec((BLK, D), lambda i: (i, 0)),
        out_shape=jax.ShapeDtypeStruct((N_NODES, D), jnp.float32),
    )(x, W)


def _tc_post_body(relu, s_ref, acc_ref, deg_ref, wn_ref, o_ref):
    acc = acc_ref[0] + acc_ref[1]
    deg = jnp.sum(deg_ref[...], axis=0)
    inv = 1.0 / jnp.clip(deg, 1.0, None)
    hn = acc * inv[:, None]
    y = s_ref[...] + jnp.dot(hn, wn_ref[...],
                             preferred_element_type=jnp.float32,
                             precision=lax.Precision.HIGHEST)
    o_ref[...] = jnp.maximum(y, 0.0) if relu else y


def _tc_post(s, accp, deg32, W_neigh, relu):
    return pl.pallas_call(
        functools.partial(_tc_post_body, relu),
        grid=(TGRID,),
        in_specs=[
            pl.BlockSpec((BLK, D), lambda i: (i, 0)),
            pl.BlockSpec((NC, BLK, D), lambda i: (0, i, 0)),
            pl.BlockSpec((NW, BLK), lambda i: (0, i)),
            pl.BlockSpec((D, D), lambda i: (0, 0)),
        ],
        out_specs=pl.BlockSpec((BLK, D), lambda i: (i, 0)),
        out_shape=jax.ShapeDtypeStruct((N_NODES, D), jnp.float32),
    )(s, accp, deg32, W_neigh)


def kernel(x, edge_index, W_self1, W_neigh1, W_self2, W_neigh2):
    ei = edge_index.astype(jnp.int32).reshape(2, NW * NCHUNK, CHUNK)
    sd = jnp.swapaxes(ei, 0, 1)
    s1 = _tc_self(x, W_self1)
    accp1, degf = _sc_agg(x, sd)
    deg32 = degf.reshape(NW, N_NODES)
    h1 = _tc_post(s1, accp1.reshape(NC, NPAD, D), deg32, W_neigh1, True)
    s2 = _tc_self(h1, W_self2)
    accp2, _ = _sc_agg(h1, sd)
    return _tc_post(s2, accp2.reshape(NC, NPAD, D), deg32, W_neigh2, False)
